# baseline (speedup 1.0000x reference)
import functools

import jax
import jax.numpy as jnp
from jax import lax
from jax.experimental import pallas as pl
from jax.experimental.pallas import tpu as pltpu
from jax.experimental.pallas import tpu_sc as plsc

N_CLASSES = 150
EPS = 0.1
ON = 1.0 - EPS
OFF = EPS / (N_CLASSES - 1)

N, H, W = 8, 224, 224
CSC = 60
CTC = N_CLASSES - CSC
RS = 8
NTASK = N * (H // RS)
CB = 30
NCHUNK = CSC // CB
NW = 32
TPW = NTASK // NW
L = 16
G = TPW * NCHUNK


def _sc_body(lab_hbm, out_hbm, lab_v, buf_v, sem0, sem1):
    cid = lax.axis_index("c")
    sid = lax.axis_index("s")
    wid = sid * 2 + cid

    on_v = jnp.full((L,), ON, jnp.float32)
    off_v = jnp.full((L,), OFF, jnp.float32)
    lane = lax.iota(jnp.int32, L)

    def fill_c(c, _):
        for p in range(2):
            for r in range(RS):
                for i in range(W // L):
                    buf_v[p, c, r, pl.ds(i * L, L)] = off_v
        return 0
    lax.fori_loop(0, CB, fill_c, 0)

    def scan_scatter(buf, labq, c0, val_v):
        for r in range(RS):
            for i in range(W // L):
                lab = lab_v[labq, r, pl.ds(i * L, L)]
                c_rel = lab - c0
                mask = c_rel.astype(jnp.uint32) < jnp.uint32(CB)
                plsc.store_scatter(
                    buf, [c_rel, jnp.full((L,), r, jnp.int32),
                          lane + (i * L)], val_v, mask=mask)

    def coords(g):
        ti = g // NCHUNK
        k = g % NCHUNK
        task = wid * TPW + ti
        n = task // (H // RS)
        h0 = (task % (H // RS)) * RS
        return ti, k, n, h0, k * CB

    def chunk_iter(g, _):
        ti, k, n, h0, c0 = coords(g)
        p = g % 2
        buf = buf_v.at[p]

        @pl.when(g >= 2)
        def _wait_and_restore():
            tip, kp, np_, h0p, c0p = coords(g - 2)
            dst = out_hbm.at[np_, pl.ds(c0p, CB), pl.ds(h0p, RS)]
            @pl.when(p == 0)
            def _():
                pltpu.make_async_copy(buf, dst, sem0).wait()
            @pl.when(p == 1)
            def _():
                pltpu.make_async_copy(buf, dst, sem1).wait()
            scan_scatter(buf, tip % 2, c0p, off_v)

        @pl.when(k == 0)
        def _load_labels():
            pltpu.sync_copy(lab_hbm.at[n, pl.ds(h0, RS)], lab_v.at[ti % 2])

        scan_scatter(buf, ti % 2, c0, on_v)

        dst = out_hbm.at[n, pl.ds(c0, CB), pl.ds(h0, RS)]
        @pl.when(p == 0)
        def _():
            pltpu.async_copy(buf, dst, sem0)
        @pl.when(p == 1)
        def _():
            pltpu.async_copy(buf, dst, sem1)
        return 0

    lax.fori_loop(0, G, chunk_iter, 0)

    ti, k, n, h0, c0 = coords(G - 2)
    pltpu.make_async_copy(
        buf_v.at[(G - 2) % 2],
        out_hbm.at[n, pl.ds(c0, CB), pl.ds(h0, RS)], sem0).wait()
    ti, k, n, h0, c0 = coords(G - 1)
    pltpu.make_async_copy(
        buf_v.at[(G - 1) % 2],
        out_hbm.at[n, pl.ds(c0, CB), pl.ds(h0, RS)], sem1).wait()


def _sc_part(lab):
    f = functools.partial(
        pl.kernel,
        mesh=plsc.VectorSubcoreMesh(core_axis_name="c", subcore_axis_name="s"),
        out_type=jax.ShapeDtypeStruct((N, CSC, H, W), jnp.float32),
        scratch_types=[
            pltpu.VMEM((2, RS, W), jnp.int32),
            pltpu.VMEM((2, CB, RS, W), jnp.float32),
            pltpu.SemaphoreType.DMA,
            pltpu.SemaphoreType.DMA,
        ],
        compiler_params=pltpu.CompilerParams(needs_layout_passes=False),
    )(_sc_body)
    return f(lab)


CBT = 45


def _tc_body(lab_ref, out_ref):
    j = pl.program_id(1)
    lab = lab_ref[0]
    cls = jax.lax.broadcasted_iota(jnp.int32, (CBT, H, W), 0) + (
        CSC + j * CBT)
    out_ref[0] = jnp.where(cls == lab, ON, OFF)


def _tc_part(lab):
    return pl.pallas_call(
        _tc_body,
        grid=(N, pl.cdiv(CTC, CBT)),
        in_specs=[pl.BlockSpec((1, 1, H, W), lambda n, j: (n, 0, 0, 0))],
        out_specs=pl.BlockSpec((1, CBT, H, W), lambda n, j: (n, j, 0, 0)),
        out_shape=jax.ShapeDtypeStruct((N, CTC, H, W), jnp.float32),
    )(lab.reshape(N, 1, H, W))


def kernel(labels):
    lab = labels.astype(jnp.int32)
    out_sc = _sc_part(lab)
    out_tc = _tc_part(lab)
    return jnp.concatenate([out_sc, out_tc], axis=1)


# speedup vs baseline: 2.2625x; 2.2625x over previous
import functools

import jax
import jax.numpy as jnp
from jax import lax
from jax.experimental import pallas as pl
from jax.experimental.pallas import tpu as pltpu
from jax.experimental.pallas import tpu_sc as plsc

N_CLASSES = 150
EPS = 0.1
ON = 1.0 - EPS
OFF = EPS / (N_CLASSES - 1)

N, H, W = 8, 224, 224
RS = 8
NTASK = N * (H // RS)
CB = 30
NCHUNK = N_CLASSES // CB
NW = 32
TPW = NTASK // NW
L = 16
G = TPW * NCHUNK


def _sc_body(lab_hbm, out_hbm, lab_v, buf_v, sem0, sem1):
    cid = lax.axis_index("c")
    sid = lax.axis_index("s")
    wid = sid * 2 + cid

    on_v = jnp.full((L,), ON, jnp.float32)
    off_v = jnp.full((L,), OFF, jnp.float32)
    lane = lax.iota(jnp.int32, L)

    def fill_buf(p):
        def fill_c(c, _):
            for r in range(RS):
                for i in range(W // L):
                    buf_v[p, c, r, pl.ds(i * L, L)] = off_v
            return 0
        lax.fori_loop(0, CB, fill_c, 0)

    def scan_scatter(buf, labq, c0, val_v):
        for r in range(RS):
            for i in range(W // L):
                lab = lab_v[labq, r, pl.ds(i * L, L)]
                c_rel = lab - c0
                mask = c_rel.astype(jnp.uint32) < jnp.uint32(CB)
                plsc.store_scatter(
                    buf, [c_rel, jnp.full((L,), r, jnp.int32),
                          lane + (i * L)], val_v, mask=mask)

    def coords(g):
        ti = g // NCHUNK
        k = g % NCHUNK
        task = wid * TPW + ti
        n = task // (H // RS)
        h0 = (task % (H // RS)) * RS
        return ti, k, n, h0, k * CB

    def chunk_iter(g, _):
        ti, k, n, h0, c0 = coords(g)
        p = g % 2
        buf = buf_v.at[p]

        @pl.when(k == 0)
        def _load_labels():
            pltpu.sync_copy(lab_hbm.at[n, pl.ds(h0, RS)], lab_v.at[ti % 2])

        @pl.when(g >= 2)
        def _wait_and_restore():
            tip, kp, np_, h0p, c0p = coords(g - 2)
            dst = out_hbm.at[np_, pl.ds(c0p, CB), pl.ds(h0p, RS)]
            @pl.when(p == 0)
            def _():
                pltpu.make_async_copy(buf, dst, sem0).wait()
            @pl.when(p == 1)
            def _():
                pltpu.make_async_copy(buf, dst, sem1).wait()
            scan_scatter(buf, tip % 2, c0p, off_v)

        scan_scatter(buf, ti % 2, c0, on_v)

        dst = out_hbm.at[n, pl.ds(c0, CB), pl.ds(h0, RS)]
        @pl.when(p == 0)
        def _():
            pltpu.async_copy(buf, dst, sem0)
        @pl.when(p == 1)
        def _():
            pltpu.async_copy(buf, dst, sem1)
        return 0

    fill_buf(0)
    chunk_iter(jnp.int32(0), 0)
    fill_buf(1)
    lax.fori_loop(1, G, chunk_iter, 0)

    ti, k, n, h0, c0 = coords(G - 2)
    pltpu.make_async_copy(
        buf_v.at[(G - 2) % 2],
        out_hbm.at[n, pl.ds(c0, CB), pl.ds(h0, RS)], sem0).wait()
    ti, k, n, h0, c0 = coords(G - 1)
    pltpu.make_async_copy(
        buf_v.at[(G - 1) % 2],
        out_hbm.at[n, pl.ds(c0, CB), pl.ds(h0, RS)], sem1).wait()


def kernel(labels):
    lab = labels.astype(jnp.int32)
    f = functools.partial(
        pl.kernel,
        mesh=plsc.VectorSubcoreMesh(core_axis_name="c", subcore_axis_name="s"),
        out_type=jax.ShapeDtypeStruct((N, N_CLASSES, H, W), jnp.float32),
        scratch_types=[
            pltpu.VMEM((2, RS, W), jnp.int32),
            pltpu.VMEM((2, CB, RS, W), jnp.float32),
            pltpu.SemaphoreType.DMA,
            pltpu.SemaphoreType.DMA,
        ],
        compiler_params=pltpu.CompilerParams(needs_layout_passes=False),
    )(_sc_body)
    return f(lab)
